# baseline (speedup 1.0000x reference)
import jax
import jax.numpy as jnp
from jax import lax
from jax.experimental import pallas as pl
from jax.experimental.pallas import tpu as pltpu
from jax.experimental.pallas import tpu_sc as plsc

N = 10000
DIM = 128
HID = 256
NC = 2
NS = 16
NW = NC * NS
LANES = 16
CHUNK = 32
N_PAD = 10112
ROWS_PER_TILE = N_PAD // NS
E1 = 320000
E3 = 100000
E1_PAD = 316 * NW * CHUNK
E3H_PAD = 100 * NW * CHUNK
E3F_PAD = 2 * E3H_PAD

_f32 = jnp.float32


def _mesh():
    return plsc.VectorSubcoreMesh(
        core_axis_name="c", subcore_axis_name="s", num_cores=NC, num_subcores=NS
    )


def _sc_conv(e_pad, p_rows):
    iters = e_pad // (NW * CHUNK)
    assert iters % 2 == 0
    per_tile = e_pad // NW
    out_type = jax.ShapeDtypeStruct((NC, N_PAD, DIM), _f32)
    scratch = [
        pltpu.VMEM((per_tile,), jnp.int32),
        pltpu.VMEM((per_tile,), jnp.int32),
        pltpu.VMEM((CHUNK, DIM), _f32),
        pltpu.VMEM((CHUNK, DIM), _f32),
        pltpu.VMEM((CHUNK, DIM), _f32),
        pltpu.VMEM((CHUNK, DIM), _f32),
        pltpu.VMEM((CHUNK, DIM), _f32),
        pltpu.VMEM((CHUNK, DIM), _f32),
        pltpu.SemaphoreType.DMA,
        pltpu.SemaphoreType.DMA,
        pltpu.SemaphoreType.DMA,
        pltpu.SemaphoreType.DMA,
        pltpu.SemaphoreType.DMA,
        pltpu.SemaphoreType.DMA,
        pltpu.SemaphoreType.DMA,
        pltpu.SemaphoreType.DMA,
        pltpu.VMEM_SHARED((N_PAD, DIM), _f32),
    ]

    def body(src_hbm, dst_hbm, s_hbm, d_hbm, p_hbm, agg_hbm, *rest):
        *bufs, aggs = rest
        srcall = bufs[0]
        dstall = bufs[1]
        srows = bufs[2:4]
        drows = bufs[4:6]
        prows = bufs[6:8]
        sems = bufs[8:14]
        semsc = bufs[14:16]
        cid = lax.axis_index("c")
        sid = lax.axis_index("s")
        w = sid * NC + cid
        zero16 = jnp.zeros((LANES,), _f32)

        pltpu.sync_copy(src_hbm.at[pl.ds(w * per_tile, per_tile)], srcall)
        pltpu.sync_copy(dst_hbm.at[pl.ds(w * per_tile, per_tile)], dstall)

        def zrow(i, carry):
            for j in range(DIM // LANES):
                prows[0][i, pl.ds(j * LANES, LANES)] = zero16
            return carry

        lax.fori_loop(0, CHUNK, zrow, 0)
        base = sid * ROWS_PER_TILE
        for k in range(ROWS_PER_TILE // CHUNK):
            pltpu.sync_copy(prows[0], aggs.at[pl.ds(base + k * CHUNK, CHUNK)])
        rem = ROWS_PER_TILE % CHUNK
        if rem:
            pltpu.sync_copy(
                prows[0].at[pl.ds(0, rem)],
                aggs.at[pl.ds(base + (ROWS_PER_TILE // CHUNK) * CHUNK, rem)],
            )
        plsc.subcore_barrier()

        def p_offset(off):
            if p_rows == e_pad:
                return off
            return jnp.where(off >= p_rows, off - p_rows, off)

        def idx(t):
            return pl.ds(t * CHUNK, CHUNK)

        def issue(t, s):
            off = w * per_tile + t * CHUNK
            pltpu.async_copy(s_hbm.at[srcall.at[idx(t)]], srows[s], sems[3 * s])
            pltpu.async_copy(d_hbm.at[dstall.at[idx(t)]], drows[s], sems[3 * s + 1])
            pltpu.async_copy(
                p_hbm.at[pl.ds(p_offset(off), CHUNK)], prows[s], sems[3 * s + 2]
            )

        def compute(t, s):
            pltpu.make_async_copy(
                s_hbm.at[srcall.at[idx(t)]], srows[s], sems[3 * s]
            ).wait()
            pltpu.make_async_copy(
                d_hbm.at[dstall.at[idx(t)]], drows[s], sems[3 * s + 1]
            ).wait()
            pltpu.make_async_copy(
                p_hbm.at[pl.ds(0, CHUNK)], prows[s], sems[3 * s + 2]
            ).wait()

            def edge_body(e, c2):
                for j in range(DIM // LANES):
                    sl = pl.ds(j * LANES, LANES)
                    prows[s][e, sl] = jnp.maximum(
                        srows[s][e, sl] + drows[s][e, sl] + prows[s][e, sl], 0.0
                    )
                return c2

            lax.fori_loop(0, CHUNK, edge_body, 0)
            pltpu.async_copy(
                prows[s], aggs.at[dstall.at[idx(t)]], semsc[s], add=True
            )

        def wait_scatter(t, s):
            pltpu.make_async_copy(
                prows[s], aggs.at[dstall.at[idx(t)]], semsc[s]
            ).wait()

        issue(0, 0)
        issue(1, 1)

        def pair_body(k, carry):
            compute(2 * k, 0)
            compute(2 * k + 1, 1)
            wait_scatter(2 * k, 0)

            @pl.when(2 * k + 2 < iters)
            def _():
                issue(2 * k + 2, 0)

            wait_scatter(2 * k + 1, 1)

            @pl.when(2 * k + 3 < iters)
            def _():
                issue(2 * k + 3, 1)

            return carry

        lax.fori_loop(0, iters // 2, pair_body, 0)
        plsc.subcore_barrier()
        pltpu.sync_copy(
            aggs.at[pl.ds(base, ROWS_PER_TILE)],
            agg_hbm.at[cid, pl.ds(base, ROWS_PER_TILE)],
        )

    return pl.kernel(
        body,
        out_type=out_type,
        mesh=_mesh(),
        scratch_types=tuple(scratch),
        compiler_params=pltpu.CompilerParams(needs_layout_passes=False),
        name="sc_conv",
    )


def _sc_count():
    pt1 = E1_PAD // NW
    pt3 = E3F_PAD // NW
    it1 = pt1 // CHUNK
    it3 = pt3 // CHUNK
    scratch = (
        pltpu.VMEM((pt1,), jnp.int32),
        pltpu.VMEM((pt3,), jnp.int32),
        pltpu.VMEM((N_PAD,), _f32),
    )

    def body(dst1_hbm, dst3_hbm, cnt1_hbm, cnt3_hbm, d1all, d3all, cntv):
        cid = lax.axis_index("c")
        sid = lax.axis_index("s")
        w = sid * NC + cid
        zero16 = jnp.zeros((LANES,), _f32)
        ones16 = jnp.full((LANES,), 1.0, _f32)
        pltpu.sync_copy(dst1_hbm.at[pl.ds(w * pt1, pt1)], d1all)
        pltpu.sync_copy(dst3_hbm.at[pl.ds(w * pt3, pt3)], d3all)

        def zcnt(i, carry):
            cntv[pl.ds(i * LANES, LANES)] = zero16
            return carry

        def count_loop(dall, iters_n):
            def cb(t, carry):
                for k in range(CHUNK // LANES):
                    plsc.addupdate_scatter(
                        cntv,
                        [dall[pl.ds(t * CHUNK + k * LANES, LANES)]],
                        ones16,
                    )
                return carry
            lax.fori_loop(0, iters_n, cb, 0)

        lax.fori_loop(0, N_PAD // LANES, zcnt, 0)
        count_loop(d1all, it1)
        pltpu.sync_copy(cntv, cnt1_hbm.at[w])
        lax.fori_loop(0, N_PAD // LANES, zcnt, 0)
        count_loop(d3all, it3)
        pltpu.sync_copy(cntv, cnt3_hbm.at[w])

    return pl.kernel(
        body,
        out_type=(
            jax.ShapeDtypeStruct((NW, N_PAD), _f32),
            jax.ShapeDtypeStruct((NW, N_PAD), _f32),
        ),
        mesh=_mesh(),
        scratch_types=scratch,
        compiler_params=pltpu.CompilerParams(needs_layout_passes=False),
        name="sc_count",
    )


def _sc_head():
    iters = E3H_PAD // (NW * CHUNK)
    assert iters % 2 == 0
    per_tile = E3H_PAD // NW
    scratch = (
        pltpu.VMEM((per_tile,), jnp.int32),
        pltpu.VMEM((per_tile,), jnp.int32),
        pltpu.VMEM((CHUNK, DIM), _f32),
        pltpu.VMEM((CHUNK, DIM), _f32),
        pltpu.VMEM((CHUNK, DIM), _f32),
        pltpu.VMEM((CHUNK, DIM), _f32),
        pltpu.VMEM((CHUNK, DIM), _f32),
        pltpu.VMEM((CHUNK, DIM), _f32),
        pltpu.VMEM((DIM,), _f32),
        pltpu.VMEM((per_tile * LANES,), _f32),
        pltpu.SemaphoreType.DMA,
        pltpu.SemaphoreType.DMA,
        pltpu.SemaphoreType.DMA,
        pltpu.SemaphoreType.DMA,
        pltpu.SemaphoreType.DMA,
        pltpu.SemaphoreType.DMA,
    )

    def body(src_hbm, dst_hbm, a_hbm, b_hbm, c_hbm, w2_hbm, y_hbm, *bufs):
        srcall = bufs[0]
        dstall = bufs[1]
        srows = bufs[2:4]
        drows = bufs[4:6]
        prows = bufs[6:8]
        w2v = bufs[8]
        accb = bufs[9]
        sems = bufs[10:16]
        cid = lax.axis_index("c")
        sid = lax.axis_index("s")
        w = sid * NC + cid
        ebase = w * per_tile
        pltpu.sync_copy(w2_hbm, w2v)
        pltpu.sync_copy(src_hbm.at[pl.ds(ebase, per_tile)], srcall)
        pltpu.sync_copy(dst_hbm.at[pl.ds(ebase, per_tile)], dstall)

        def idx(t):
            return pl.ds(t * CHUNK, CHUNK)

        def issue(t, s):
            off = ebase + t * CHUNK
            pltpu.async_copy(a_hbm.at[srcall.at[idx(t)]], srows[s], sems[3 * s])
            pltpu.async_copy(b_hbm.at[dstall.at[idx(t)]], drows[s], sems[3 * s + 1])
            pltpu.async_copy(c_hbm.at[pl.ds(off, CHUNK)], prows[s], sems[3 * s + 2])

        def process(t, s):
            pltpu.make_async_copy(
                a_hbm.at[srcall.at[idx(t)]], srows[s], sems[3 * s]
            ).wait()
            pltpu.make_async_copy(
                b_hbm.at[dstall.at[idx(t)]], drows[s], sems[3 * s + 1]
            ).wait()
            pltpu.make_async_copy(
                c_hbm.at[pl.ds(0, CHUNK)], prows[s], sems[3 * s + 2]
            ).wait()

            def edge_body(e, c2):
                acc = jnp.zeros((LANES,), _f32)
                for j in range(DIM // LANES):
                    sl = pl.ds(j * LANES, LANES)
                    z = jnp.maximum(
                        srows[s][e, sl] + drows[s][e, sl] + prows[s][e, sl], 0.0
                    )
                    acc = acc + z * w2v[sl]
                accb[pl.ds((t * CHUNK + e) * LANES, LANES)] = acc
                return c2

            lax.fori_loop(0, CHUNK, edge_body, 0)

        issue(0, 0)
        issue(1, 1)

        def pair_body(k, carry):
            process(2 * k, 0)

            @pl.when(2 * k + 2 < iters)
            def _():
                issue(2 * k + 2, 0)

            process(2 * k + 1, 1)

            @pl.when(2 * k + 3 < iters)
            def _():
                issue(2 * k + 3, 1)

            return carry

        lax.fori_loop(0, iters // 2, pair_body, 0)
        pltpu.sync_copy(accb, y_hbm.at[pl.ds(ebase * LANES, per_tile * LANES)])

    return pl.kernel(
        body,
        out_type=jax.ShapeDtypeStruct((E3H_PAD * LANES,), _f32),
        mesh=_mesh(),
        scratch_types=scratch,
        compiler_params=pltpu.CompilerParams(needs_layout_passes=False),
        name="sc_head",
    )


def _prologue_body(x_ref, g1_ref, b1_ref, w1_ref, bb1_ref, g2_ref, b2_ref,
                   w2_ref, bb2_ref, wms_ref, wmd_ref, bm_ref,
                   out_ref, s_ref, d_ref):
    x = x_ref[...]
    inv_n = 1.0 / N
    m = jnp.sum(x, axis=0, keepdims=True) * inv_n
    v = jnp.sum(x * x, axis=0, keepdims=True) * inv_n - m * m
    xn = (x - m) / jnp.sqrt(v + 1e-5) * g1_ref[...] + b1_ref[...]
    h = jnp.maximum(
        jnp.dot(xn, w1_ref[...], preferred_element_type=_f32) + bb1_ref[...], 0.0
    )
    rowmask = lax.broadcasted_iota(jnp.int32, (N_PAD, 1), 0) < N
    h = jnp.where(rowmask, h, 0.0)
    m2 = jnp.sum(h, axis=0, keepdims=True) * inv_n
    v2 = jnp.sum(h * h, axis=0, keepdims=True) * inv_n - m2 * m2
    hn = (h - m2) / jnp.sqrt(v2 + 1e-5) * g2_ref[...] + b2_ref[...]
    out = jnp.maximum(
        jnp.dot(hn, w2_ref[...], preferred_element_type=_f32) + bb2_ref[...], 0.0
    )
    out_ref[...] = out
    s_ref[...] = jnp.dot(out, wms_ref[...], preferred_element_type=_f32)
    d_ref[...] = jnp.dot(out, wmd_ref[...], preferred_element_type=_f32) + bm_ref[...]


def _tc_prologue(x_pad, g1, b1, W1, bb1, g2, b2, W2, bb2, wms, wmd, bm):
    shp = jax.ShapeDtypeStruct((N_PAD, DIM), _f32)
    return pl.pallas_call(
        _prologue_body,
        out_shape=(shp, shp, shp),
    )(x_pad, g1, b1, W1, bb1, g2, b2, W2, bb2, wms, wmd, bm)


def _make_post_body(first, d_bias):
    def body(*refs):
        i = 0
        out_prev_ref = refs[i]; i += 1
        agg_ref = refs[i]; i += 1
        cnt_ref = refs[i]; i += 1
        wu_ref = refs[i]; i += 1
        bu_ref = refs[i]; i += 1
        wms_ref = refs[i]; i += 1
        wmd_ref = refs[i]; i += 1
        bm_ref = refs[i] if d_bias else None
        i += 1 if d_bias else 0
        out_new_ref = refs[i]; i += 1
        s_ref = refs[i]; i += 1
        d_ref = refs[i]; i += 1
        inv_ref = refs[i] if first else None

        a = agg_ref[0] + agg_ref[1]
        if first:
            cs = jnp.sum(cnt_ref[...], axis=1, keepdims=True)
            inv = 1.0 / jnp.maximum(cs, 1.0)
            inv_ref[...] = inv
        else:
            inv = cnt_ref[...]
        upd = jnp.dot(a, wu_ref[...], preferred_element_type=_f32) * inv
        out = jnp.maximum(out_prev_ref[...] + upd + bu_ref[...], 0.0)
        out_new_ref[...] = out
        s_ref[...] = jnp.dot(out, wms_ref[...], preferred_element_type=_f32)
        d = jnp.dot(out, wmd_ref[...], preferred_element_type=_f32)
        if d_bias:
            d = d + bm_ref[...]
        d_ref[...] = d
    return body


def _tc_post(first, d_bias, out_prev, agg, cnt_or_inv, wu, bu, wms, wmd, bm=None):
    shp = jax.ShapeDtypeStruct((N_PAD, DIM), _f32)
    outs = [shp, shp, shp]
    if first:
        outs.append(jax.ShapeDtypeStruct((N_PAD, 1), _f32))
    args = [out_prev, agg, cnt_or_inv, wu, bu, wms, wmd]
    if d_bias:
        args.append(bm)
    return pl.pallas_call(
        _make_post_body(first, d_bias),
        out_shape=tuple(outs),
    )(*args)


def _proj_body_factory(n_out, with_bias):
    def body(*refs):
        e = refs[0][...]
        for k in range(n_out):
            w = refs[1 + k][...]
            o = jnp.dot(e, w, preferred_element_type=_f32)
            if with_bias and k == n_out - 1:
                o = o + refs[1 + n_out][...]
            refs[-(n_out - k)][...] = o
    return body


def _tc_proj(ea_pad, weights, bias=None, blk=4096):
    e_pad, k_in = ea_pad.shape
    n_out = len(weights)
    grid = e_pad // blk
    in_specs = [pl.BlockSpec((blk, k_in), lambda i: (i, 0))]
    for _ in weights:
        in_specs.append(pl.BlockSpec((k_in, DIM), lambda i: (0, 0)))
    args = [ea_pad] + list(weights)
    if bias is not None:
        in_specs.append(pl.BlockSpec((DIM,), lambda i: (0,)))
        args.append(bias)
    out_shape = tuple(jax.ShapeDtypeStruct((e_pad, DIM), _f32) for _ in range(n_out))
    out_specs = tuple(pl.BlockSpec((blk, DIM), lambda i: (i, 0)) for _ in range(n_out))
    return pl.pallas_call(
        _proj_body_factory(n_out, bias is not None),
        grid=(grid,),
        in_specs=in_specs,
        out_specs=out_specs,
        out_shape=out_shape,
    )(*args)


def _reduce16_body(y_ref, b_ref, o_ref):
    o_ref[...] = jnp.sum(y_ref[...], axis=1, keepdims=True) + b_ref[0]


def _tc_reduce16(y16, bh2, blk=4096):
    grid = E3H_PAD // blk
    return pl.pallas_call(
        _reduce16_body,
        grid=(grid,),
        in_specs=[
            pl.BlockSpec((blk, LANES), lambda i: (i, 0)),
            pl.BlockSpec(memory_space=pltpu.SMEM),
        ],
        out_specs=pl.BlockSpec((blk, 1), lambda i: (i, 0)),
        out_shape=jax.ShapeDtypeStruct((E3H_PAD, 1), _f32),
    )(y16, bh2)


def kernel(x, edge_index, edge_attr, edge_index3, edge_attr3, edge_attr4, batch,
           bn1_g, bn1_b, W1, b1, bn2_g, bn2_b, W2, b2,
           c1_Wm, c1_bm, c1_Wu, c1_bu, c2_Wm, c2_bm, c2_Wu, c2_bu,
           Wh1, bh1, Wh2, bh2):
    x_pad = jnp.zeros((N_PAD, DIM), _f32).at[:N].set(x)
    src1 = jnp.pad(edge_index[0], (0, E1_PAD - E1))
    dst1 = jnp.pad(edge_index[1], (0, E1_PAD - E1), constant_values=N)
    ea1 = jnp.pad(edge_attr, ((0, E1_PAD - E1), (0, 0)))
    s3 = edge_index3[0]
    d3 = edge_index3[1]
    s3p = jnp.pad(s3, (0, E3H_PAD - E3))
    d3p = jnp.pad(d3, (0, E3H_PAD - E3))
    s3n = jnp.pad(s3, (0, E3H_PAD - E3), constant_values=N)
    d3n = jnp.pad(d3, (0, E3H_PAD - E3), constant_values=N)
    src3f = jnp.concatenate([s3p, d3p])
    dst3f = jnp.concatenate([d3n, s3n])
    temp = jnp.concatenate([edge_attr3, edge_attr4], axis=1)
    temp_pad = jnp.pad(temp, ((0, E3H_PAD - E3), (0, 0)))

    P1a, P1b = _tc_proj(ea1, (c1_Wm[0, 2 * DIM:], c1_Wm[1, 2 * DIM:]))
    P2a, P2b, Ch = _tc_proj(
        temp_pad, (c2_Wm[0, 2 * DIM:], c2_Wm[1, 2 * DIM:], Wh1[2 * DIM:]), bias=bh1
    )

    out0, S, D = _tc_prologue(
        x_pad, bn1_g, bn1_b, W1, b1, bn2_g, bn2_b, W2, b2,
        c1_Wm[0, :DIM], c1_Wm[0, DIM:2 * DIM], c1_bm[0]
    )

    cntp, cntp3 = _sc_count()(dst1, dst3f)

    agg = _sc_conv(E1_PAD, E1_PAD)(src1, dst1, S, D, P1a)
    out1, S, D, inv1 = _tc_post(
        True, True, out0, agg, cntp.T, c1_Wu[0], c1_bu[0],
        c1_Wm[1, :DIM], c1_Wm[1, DIM:2 * DIM], c1_bm[1]
    )

    agg = _sc_conv(E1_PAD, E1_PAD)(src1, dst1, S, D, P1b)
    out2, S, D = _tc_post(
        False, True, out1, agg, inv1, c1_Wu[1], c1_bu[1],
        c2_Wm[0, :DIM], c2_Wm[0, DIM:2 * DIM], c2_bm[0]
    )

    agg = _sc_conv(E3F_PAD, E3H_PAD)(src3f, dst3f, S, D, P2a)
    out3, S, D, inv3 = _tc_post(
        True, True, out2, agg, cntp3.T, c2_Wu[0], c2_bu[0],
        c2_Wm[1, :DIM], c2_Wm[1, DIM:2 * DIM], c2_bm[1]
    )

    agg = _sc_conv(E3F_PAD, E3H_PAD)(src3f, dst3f, S, D, P2b)
    _, A, B = _tc_post(
        False, False, out3, agg, inv3, c2_Wu[1], c2_bu[1],
        Wh1[:DIM], Wh1[DIM:2 * DIM]
    )

    y16 = _sc_head()(s3p, d3p, A, B, Ch, Wh2[:, 0]).reshape(E3H_PAD, LANES)
    ycol = _tc_reduce16(y16, bh2)
    return ycol[:E3, 0]


# speedup vs baseline: 1.1116x; 1.1116x over previous
import jax
import jax.numpy as jnp
from jax import lax
from jax.experimental import pallas as pl
from jax.experimental.pallas import tpu as pltpu
from jax.experimental.pallas import tpu_sc as plsc

N = 10000
DIM = 128
HID = 256
NC = 2
NS = 16
NW = NC * NS
LANES = 16
CHUNK = 32
N_PAD = 10112
ROWS_PER_TILE = N_PAD // NS
E1 = 320000
E3 = 100000
E1_PAD = 316 * NW * CHUNK
E3H_PAD = 100 * NW * CHUNK
E3F_PAD = 2 * E3H_PAD

_f32 = jnp.float32


def _mesh():
    return plsc.VectorSubcoreMesh(
        core_axis_name="c", subcore_axis_name="s", num_cores=NC, num_subcores=NS
    )


def _sc_conv(e_pad, p_rows):
    iters = e_pad // (NW * CHUNK)
    assert iters % 2 == 0
    per_tile = e_pad // NW
    out_type = jax.ShapeDtypeStruct((NC, N_PAD, DIM), _f32)
    scratch = [
        pltpu.VMEM((per_tile,), jnp.int32),
        pltpu.VMEM((per_tile,), jnp.int32),
        pltpu.VMEM((CHUNK, DIM), _f32),
        pltpu.VMEM((CHUNK, DIM), _f32),
        pltpu.VMEM((CHUNK, DIM), _f32),
        pltpu.VMEM((CHUNK, DIM), _f32),
        pltpu.VMEM((CHUNK, DIM), _f32),
        pltpu.VMEM((CHUNK, DIM), _f32),
        pltpu.SemaphoreType.DMA,
        pltpu.SemaphoreType.DMA,
        pltpu.SemaphoreType.DMA,
        pltpu.SemaphoreType.DMA,
        pltpu.SemaphoreType.DMA,
        pltpu.SemaphoreType.DMA,
        pltpu.SemaphoreType.DMA,
        pltpu.SemaphoreType.DMA,
        pltpu.VMEM_SHARED((N_PAD, DIM), _f32),
    ]

    def body(src_hbm, dst_hbm, s_hbm, d_hbm, p_hbm, agg_hbm, *rest):
        *bufs, aggs = rest
        srcall = bufs[0]
        dstall = bufs[1]
        srows = bufs[2:4]
        drows = bufs[4:6]
        prows = bufs[6:8]
        sems = bufs[8:14]
        semsc = bufs[14:16]
        cid = lax.axis_index("c")
        sid = lax.axis_index("s")
        w = sid * NC + cid
        zero16 = jnp.zeros((LANES,), _f32)

        pltpu.sync_copy(src_hbm.at[pl.ds(w * per_tile, per_tile)], srcall)
        pltpu.sync_copy(dst_hbm.at[pl.ds(w * per_tile, per_tile)], dstall)

        def zrow(i, carry):
            for j in range(DIM // LANES):
                prows[0][i, pl.ds(j * LANES, LANES)] = zero16
            return carry

        lax.fori_loop(0, CHUNK, zrow, 0)
        base = sid * ROWS_PER_TILE
        for k in range(ROWS_PER_TILE // CHUNK):
            pltpu.sync_copy(prows[0], aggs.at[pl.ds(base + k * CHUNK, CHUNK)])
        rem = ROWS_PER_TILE % CHUNK
        if rem:
            pltpu.sync_copy(
                prows[0].at[pl.ds(0, rem)],
                aggs.at[pl.ds(base + (ROWS_PER_TILE // CHUNK) * CHUNK, rem)],
            )
        plsc.subcore_barrier()

        def p_offset(off):
            if p_rows == e_pad:
                return off
            return jnp.where(off >= p_rows, off - p_rows, off)

        def idx(t):
            return pl.ds(t * CHUNK, CHUNK)

        def issue(t, s):
            off = w * per_tile + t * CHUNK
            pltpu.async_copy(s_hbm.at[srcall.at[idx(t)]], srows[s], sems[3 * s])
            pltpu.async_copy(d_hbm.at[dstall.at[idx(t)]], drows[s], sems[3 * s + 1])
            pltpu.async_copy(
                p_hbm.at[pl.ds(p_offset(off), CHUNK)], prows[s], sems[3 * s + 2]
            )

        def compute(t, s):
            pltpu.make_async_copy(
                s_hbm.at[srcall.at[idx(t)]], srows[s], sems[3 * s]
            ).wait()
            pltpu.make_async_copy(
                d_hbm.at[dstall.at[idx(t)]], drows[s], sems[3 * s + 1]
            ).wait()
            pltpu.make_async_copy(
                p_hbm.at[pl.ds(0, CHUNK)], prows[s], sems[3 * s + 2]
            ).wait()

            def edge_body(e, c2):
                for j in range(DIM // LANES):
                    sl = pl.ds(j * LANES, LANES)
                    prows[s][e, sl] = jnp.maximum(
                        srows[s][e, sl] + drows[s][e, sl] + prows[s][e, sl], 0.0
                    )
                return c2

            lax.fori_loop(0, CHUNK, edge_body, 0)
            pltpu.sync_copy(prows[s], aggs.at[dstall.at[idx(t)]], add=True)

        issue(0, 0)
        issue(1, 1)

        def pair_body(k, carry):
            compute(2 * k, 0)

            @pl.when(2 * k + 2 < iters)
            def _():
                issue(2 * k + 2, 0)

            compute(2 * k + 1, 1)

            @pl.when(2 * k + 3 < iters)
            def _():
                issue(2 * k + 3, 1)

            return carry

        lax.fori_loop(0, iters // 2, pair_body, 0)
        plsc.subcore_barrier()
        pltpu.sync_copy(
            aggs.at[pl.ds(base, ROWS_PER_TILE)],
            agg_hbm.at[cid, pl.ds(base, ROWS_PER_TILE)],
        )

    return pl.kernel(
        body,
        out_type=out_type,
        mesh=_mesh(),
        scratch_types=tuple(scratch),
        compiler_params=pltpu.CompilerParams(needs_layout_passes=False),
        name="sc_conv",
    )


def _sc_count():
    pt1 = E1_PAD // NW
    pt3 = E3F_PAD // NW
    it1 = pt1 // CHUNK
    it3 = pt3 // CHUNK
    scratch = (
        pltpu.VMEM((pt1,), jnp.int32),
        pltpu.VMEM((pt3,), jnp.int32),
        pltpu.VMEM((N_PAD,), _f32),
    )

    def body(dst1_hbm, dst3_hbm, cnt1_hbm, cnt3_hbm, d1all, d3all, cntv):
        cid = lax.axis_index("c")
        sid = lax.axis_index("s")
        w = sid * NC + cid
        zero16 = jnp.zeros((LANES,), _f32)
        ones16 = jnp.full((LANES,), 1.0, _f32)
        pltpu.sync_copy(dst1_hbm.at[pl.ds(w * pt1, pt1)], d1all)
        pltpu.sync_copy(dst3_hbm.at[pl.ds(w * pt3, pt3)], d3all)

        def zcnt(i, carry):
            cntv[pl.ds(i * LANES, LANES)] = zero16
            return carry

        def count_loop(dall, iters_n):
            def cb(t, carry):
                for k in range(CHUNK // LANES):
                    plsc.addupdate_scatter(
                        cntv,
                        [dall[pl.ds(t * CHUNK + k * LANES, LANES)]],
                        ones16,
                    )
                return carry
            lax.fori_loop(0, iters_n, cb, 0)

        lax.fori_loop(0, N_PAD // LANES, zcnt, 0)
        count_loop(d1all, it1)
        pltpu.sync_copy(cntv, cnt1_hbm.at[w])
        lax.fori_loop(0, N_PAD // LANES, zcnt, 0)
        count_loop(d3all, it3)
        pltpu.sync_copy(cntv, cnt3_hbm.at[w])

    return pl.kernel(
        body,
        out_type=(
            jax.ShapeDtypeStruct((NW, N_PAD), _f32),
            jax.ShapeDtypeStruct((NW, N_PAD), _f32),
        ),
        mesh=_mesh(),
        scratch_types=scratch,
        compiler_params=pltpu.CompilerParams(needs_layout_passes=False),
        name="sc_count",
    )


def _sc_head():
    iters = E3H_PAD // (NW * CHUNK)
    assert iters % 2 == 0
    per_tile = E3H_PAD // NW
    scratch = (
        pltpu.VMEM((per_tile,), jnp.int32),
        pltpu.VMEM((per_tile,), jnp.int32),
        pltpu.VMEM((CHUNK, DIM), _f32),
        pltpu.VMEM((CHUNK, DIM), _f32),
        pltpu.VMEM((CHUNK, DIM), _f32),
        pltpu.VMEM((CHUNK, DIM), _f32),
        pltpu.VMEM((CHUNK, DIM), _f32),
        pltpu.VMEM((CHUNK, DIM), _f32),
        pltpu.VMEM((DIM,), _f32),
        pltpu.VMEM((per_tile * LANES,), _f32),
        pltpu.SemaphoreType.DMA,
        pltpu.SemaphoreType.DMA,
        pltpu.SemaphoreType.DMA,
        pltpu.SemaphoreType.DMA,
        pltpu.SemaphoreType.DMA,
        pltpu.SemaphoreType.DMA,
    )

    def body(src_hbm, dst_hbm, a_hbm, b_hbm, c_hbm, w2_hbm, y_hbm, *bufs):
        srcall = bufs[0]
        dstall = bufs[1]
        srows = bufs[2:4]
        drows = bufs[4:6]
        prows = bufs[6:8]
        w2v = bufs[8]
        accb = bufs[9]
        sems = bufs[10:16]
        cid = lax.axis_index("c")
        sid = lax.axis_index("s")
        w = sid * NC + cid
        ebase = w * per_tile
        pltpu.sync_copy(w2_hbm, w2v)
        pltpu.sync_copy(src_hbm.at[pl.ds(ebase, per_tile)], srcall)
        pltpu.sync_copy(dst_hbm.at[pl.ds(ebase, per_tile)], dstall)

        def idx(t):
            return pl.ds(t * CHUNK, CHUNK)

        def issue(t, s):
            off = ebase + t * CHUNK
            pltpu.async_copy(a_hbm.at[srcall.at[idx(t)]], srows[s], sems[3 * s])
            pltpu.async_copy(b_hbm.at[dstall.at[idx(t)]], drows[s], sems[3 * s + 1])
            pltpu.async_copy(c_hbm.at[pl.ds(off, CHUNK)], prows[s], sems[3 * s + 2])

        def process(t, s):
            pltpu.make_async_copy(
                a_hbm.at[srcall.at[idx(t)]], srows[s], sems[3 * s]
            ).wait()
            pltpu.make_async_copy(
                b_hbm.at[dstall.at[idx(t)]], drows[s], sems[3 * s + 1]
            ).wait()
            pltpu.make_async_copy(
                c_hbm.at[pl.ds(0, CHUNK)], prows[s], sems[3 * s + 2]
            ).wait()

            def edge_body(e, c2):
                acc = jnp.zeros((LANES,), _f32)
                for j in range(DIM // LANES):
                    sl = pl.ds(j * LANES, LANES)
                    z = jnp.maximum(
                        srows[s][e, sl] + drows[s][e, sl] + prows[s][e, sl], 0.0
                    )
                    acc = acc + z * w2v[sl]
                accb[pl.ds((t * CHUNK + e) * LANES, LANES)] = acc
                return c2

            lax.fori_loop(0, CHUNK, edge_body, 0)

        issue(0, 0)
        issue(1, 1)

        def pair_body(k, carry):
            process(2 * k, 0)

            @pl.when(2 * k + 2 < iters)
            def _():
                issue(2 * k + 2, 0)

            process(2 * k + 1, 1)

            @pl.when(2 * k + 3 < iters)
            def _():
                issue(2 * k + 3, 1)

            return carry

        lax.fori_loop(0, iters // 2, pair_body, 0)
        pltpu.sync_copy(accb, y_hbm.at[pl.ds(ebase * LANES, per_tile * LANES)])

    return pl.kernel(
        body,
        out_type=jax.ShapeDtypeStruct((E3H_PAD * LANES,), _f32),
        mesh=_mesh(),
        scratch_types=scratch,
        compiler_params=pltpu.CompilerParams(needs_layout_passes=False),
        name="sc_head",
    )


def _prologue_body(x_ref, g1_ref, b1_ref, w1_ref, bb1_ref, g2_ref, b2_ref,
                   w2_ref, bb2_ref, wms_ref, wmd_ref, bm_ref,
                   out_ref, s_ref, d_ref):
    x = x_ref[...]
    inv_n = 1.0 / N
    m = jnp.sum(x, axis=0, keepdims=True) * inv_n
    v = jnp.sum(x * x, axis=0, keepdims=True) * inv_n - m * m
    xn = (x - m) / jnp.sqrt(v + 1e-5) * g1_ref[...] + b1_ref[...]
    h = jnp.maximum(
        jnp.dot(xn, w1_ref[...], preferred_element_type=_f32) + bb1_ref[...], 0.0
    )
    rowmask = lax.broadcasted_iota(jnp.int32, (N_PAD, 1), 0) < N
    h = jnp.where(rowmask, h, 0.0)
    m2 = jnp.sum(h, axis=0, keepdims=True) * inv_n
    v2 = jnp.sum(h * h, axis=0, keepdims=True) * inv_n - m2 * m2
    hn = (h - m2) / jnp.sqrt(v2 + 1e-5) * g2_ref[...] + b2_ref[...]
    out = jnp.maximum(
        jnp.dot(hn, w2_ref[...], preferred_element_type=_f32) + bb2_ref[...], 0.0
    )
    out_ref[...] = out
    s_ref[...] = jnp.dot(out, wms_ref[...], preferred_element_type=_f32)
    d_ref[...] = jnp.dot(out, wmd_ref[...], preferred_element_type=_f32) + bm_ref[...]


def _tc_prologue(x_pad, g1, b1, W1, bb1, g2, b2, W2, bb2, wms, wmd, bm):
    shp = jax.ShapeDtypeStruct((N_PAD, DIM), _f32)
    return pl.pallas_call(
        _prologue_body,
        out_shape=(shp, shp, shp),
    )(x_pad, g1, b1, W1, bb1, g2, b2, W2, bb2, wms, wmd, bm)


def _make_post_body(first, d_bias):
    def body(*refs):
        i = 0
        out_prev_ref = refs[i]; i += 1
        agg_ref = refs[i]; i += 1
        cnt_ref = refs[i]; i += 1
        wu_ref = refs[i]; i += 1
        bu_ref = refs[i]; i += 1
        wms_ref = refs[i]; i += 1
        wmd_ref = refs[i]; i += 1
        bm_ref = refs[i] if d_bias else None
        i += 1 if d_bias else 0
        out_new_ref = refs[i]; i += 1
        s_ref = refs[i]; i += 1
        d_ref = refs[i]; i += 1
        inv_ref = refs[i] if first else None

        a = agg_ref[0] + agg_ref[1]
        if first:
            cs = jnp.sum(cnt_ref[...], axis=1, keepdims=True)
            inv = 1.0 / jnp.maximum(cs, 1.0)
            inv_ref[...] = inv
        else:
            inv = cnt_ref[...]
        upd = jnp.dot(a, wu_ref[...], preferred_element_type=_f32) * inv
        out = jnp.maximum(out_prev_ref[...] + upd + bu_ref[...], 0.0)
        out_new_ref[...] = out
        s_ref[...] = jnp.dot(out, wms_ref[...], preferred_element_type=_f32)
        d = jnp.dot(out, wmd_ref[...], preferred_element_type=_f32)
        if d_bias:
            d = d + bm_ref[...]
        d_ref[...] = d
    return body


def _tc_post(first, d_bias, out_prev, agg, cnt_or_inv, wu, bu, wms, wmd, bm=None):
    shp = jax.ShapeDtypeStruct((N_PAD, DIM), _f32)
    outs = [shp, shp, shp]
    if first:
        outs.append(jax.ShapeDtypeStruct((N_PAD, 1), _f32))
    args = [out_prev, agg, cnt_or_inv, wu, bu, wms, wmd]
    if d_bias:
        args.append(bm)
    return pl.pallas_call(
        _make_post_body(first, d_bias),
        out_shape=tuple(outs),
    )(*args)


def _proj_body_factory(n_out, with_bias):
    def body(*refs):
        e = refs[0][...]
        for k in range(n_out):
            w = refs[1 + k][...]
            o = jnp.dot(e, w, preferred_element_type=_f32)
            if with_bias and k == n_out - 1:
                o = o + refs[1 + n_out][...]
            refs[-(n_out - k)][...] = o
    return body


def _tc_proj(ea_pad, weights, bias=None, blk=4096):
    e_pad, k_in = ea_pad.shape
    n_out = len(weights)
    grid = e_pad // blk
    in_specs = [pl.BlockSpec((blk, k_in), lambda i: (i, 0))]
    for _ in weights:
        in_specs.append(pl.BlockSpec((k_in, DIM), lambda i: (0, 0)))
    args = [ea_pad] + list(weights)
    if bias is not None:
        in_specs.append(pl.BlockSpec((DIM,), lambda i: (0,)))
        args.append(bias)
    out_shape = tuple(jax.ShapeDtypeStruct((e_pad, DIM), _f32) for _ in range(n_out))
    out_specs = tuple(pl.BlockSpec((blk, DIM), lambda i: (i, 0)) for _ in range(n_out))
    return pl.pallas_call(
        _proj_body_factory(n_out, bias is not None),
        grid=(grid,),
        in_specs=in_specs,
        out_specs=out_specs,
        out_shape=out_shape,
    )(*args)


def _reduce16_body(y_ref, b_ref, o_ref):
    o_ref[...] = jnp.sum(y_ref[...], axis=1, keepdims=True) + b_ref[0]


def _tc_reduce16(y16, bh2, blk=4096):
    grid = E3H_PAD // blk
    return pl.pallas_call(
        _reduce16_body,
        grid=(grid,),
        in_specs=[
            pl.BlockSpec((blk, LANES), lambda i: (i, 0)),
            pl.BlockSpec(memory_space=pltpu.SMEM),
        ],
        out_specs=pl.BlockSpec((blk, 1), lambda i: (i, 0)),
        out_shape=jax.ShapeDtypeStruct((E3H_PAD, 1), _f32),
    )(y16, bh2)


def kernel(x, edge_index, edge_attr, edge_index3, edge_attr3, edge_attr4, batch,
           bn1_g, bn1_b, W1, b1, bn2_g, bn2_b, W2, b2,
           c1_Wm, c1_bm, c1_Wu, c1_bu, c2_Wm, c2_bm, c2_Wu, c2_bu,
           Wh1, bh1, Wh2, bh2):
    x_pad = jnp.zeros((N_PAD, DIM), _f32).at[:N].set(x)
    src1 = jnp.pad(edge_index[0], (0, E1_PAD - E1))
    dst1 = jnp.pad(edge_index[1], (0, E1_PAD - E1), constant_values=N)
    ea1 = jnp.pad(edge_attr, ((0, E1_PAD - E1), (0, 0)))
    s3 = edge_index3[0]
    d3 = edge_index3[1]
    s3p = jnp.pad(s3, (0, E3H_PAD - E3))
    d3p = jnp.pad(d3, (0, E3H_PAD - E3))
    s3n = jnp.pad(s3, (0, E3H_PAD - E3), constant_values=N)
    d3n = jnp.pad(d3, (0, E3H_PAD - E3), constant_values=N)
    src3f = jnp.concatenate([s3p, d3p])
    dst3f = jnp.concatenate([d3n, s3n])
    temp = jnp.concatenate([edge_attr3, edge_attr4], axis=1)
    temp_pad = jnp.pad(temp, ((0, E3H_PAD - E3), (0, 0)))

    P1a, P1b = _tc_proj(ea1, (c1_Wm[0, 2 * DIM:], c1_Wm[1, 2 * DIM:]))
    P2a, P2b, Ch = _tc_proj(
        temp_pad, (c2_Wm[0, 2 * DIM:], c2_Wm[1, 2 * DIM:], Wh1[2 * DIM:]), bias=bh1
    )

    out0, S, D = _tc_prologue(
        x_pad, bn1_g, bn1_b, W1, b1, bn2_g, bn2_b, W2, b2,
        c1_Wm[0, :DIM], c1_Wm[0, DIM:2 * DIM], c1_bm[0]
    )

    cntp, cntp3 = _sc_count()(dst1, dst3f)

    agg = _sc_conv(E1_PAD, E1_PAD)(src1, dst1, S, D, P1a)
    out1, S, D, inv1 = _tc_post(
        True, True, out0, agg, cntp.T, c1_Wu[0], c1_bu[0],
        c1_Wm[1, :DIM], c1_Wm[1, DIM:2 * DIM], c1_bm[1]
    )

    agg = _sc_conv(E1_PAD, E1_PAD)(src1, dst1, S, D, P1b)
    out2, S, D = _tc_post(
        False, True, out1, agg, inv1, c1_Wu[1], c1_bu[1],
        c2_Wm[0, :DIM], c2_Wm[0, DIM:2 * DIM], c2_bm[0]
    )

    agg = _sc_conv(E3F_PAD, E3H_PAD)(src3f, dst3f, S, D, P2a)
    out3, S, D, inv3 = _tc_post(
        True, True, out2, agg, cntp3.T, c2_Wu[0], c2_bu[0],
        c2_Wm[1, :DIM], c2_Wm[1, DIM:2 * DIM], c2_bm[1]
    )

    agg = _sc_conv(E3F_PAD, E3H_PAD)(src3f, dst3f, S, D, P2b)
    _, A, B = _tc_post(
        False, False, out3, agg, inv3, c2_Wu[1], c2_bu[1],
        Wh1[:DIM], Wh1[DIM:2 * DIM]
    )

    y16 = _sc_head()(s3p, d3p, A, B, Ch, Wh2[:, 0]).reshape(E3H_PAD, LANES)
    ycol = _tc_reduce16(y16, bh2)
    return ycol[:E3, 0]
